# baseline (speedup 1.0000x reference)
import functools

import jax
import jax.numpy as jnp
from jax import lax
from jax.experimental import pallas as pl
from jax.experimental.pallas import tpu as pltpu
from jax.experimental.pallas import tpu_sc as plsc

N_NODES = 10000
N_EDGES = 320000
D_FEAT = 128

NC = 2
NS = 16
NW = NC * NS
CHUNK = 2048
N_CHUNKS = -(-N_EDGES // CHUNK)
N_FULL = N_CHUNKS - 1
TAIL = N_EDGES - N_FULL * CHUNK
TAIL_WID = N_FULL % NW
SLOTS = -(-N_CHUNKS // NW)
NODES_PAD = 10240
X_ROWS = 320


def _sc_body(cat_hbm, edge_hbm, out_hbm,
             x0_v, edge_v, ax_v, ay_v,
             acc_sx, acc_cx, acc_sy, acc_cy, sem0, sem1, semx):
    AX_ROW = NODES_PAD // 128
    AY_ROW = AX_ROW + N_EDGES // 128 + 4
    cid = lax.axis_index("c")
    sid = lax.axis_index("s")
    wid = cid * NS + sid
    sems = (sem0, sem1)

    lanes = lax.iota(jnp.int32, 16)
    zf = jnp.zeros((16,), jnp.float32)
    onef = jnp.full((16,), 1.0, jnp.float32)
    col0 = jnp.zeros((16,), jnp.int32)

    CROWS = CHUNK // 128

    def chunk_copies(k, b):
        c = wid + k * NW
        gb = c * CHUNK
        gr = c * CROWS
        rax = pl.multiple_of(AX_ROW + gr, 8)
        ray = pl.multiple_of(AY_ROW + gr, 8)
        return (
            pltpu.make_async_copy(edge_hbm.at[:, pl.ds(gb, CHUNK)],
                                  edge_v.at[b], sems[b]),
            pltpu.make_async_copy(cat_hbm.at[pl.ds(rax, CROWS), :],
                                  ax_v.at[b], sems[b]),
            pltpu.make_async_copy(cat_hbm.at[pl.ds(ray, CROWS), :],
                                  ay_v.at[b], sems[b]),
        )

    def issue(k, b):
        @pl.when(wid + k * NW < N_FULL)
        def _():
            for cp in chunk_copies(k, b):
                cp.start()

    def wait(k, b):
        @pl.when(wid + k * NW < N_FULL)
        def _():
            for cp in chunk_copies(k, b):
                cp.wait()

    issue(0, 0)
    issue(1, 1)
    xcp = pltpu.make_async_copy(cat_hbm.at[pl.ds(0, AX_ROW), :], x0_v, semx)
    xcp.start()

    def zero_body(j, carry):
        acc_sx[pl.ds(j * 16, 16)] = zf
        acc_cx[pl.ds(j * 16, 16)] = zf
        acc_sy[pl.ds(j * 16, 16)] = zf
        acc_cy[pl.ds(j * 16, 16)] = zf
        return carry

    lax.fori_loop(0, NODES_PAD // 16, zero_body, 0, unroll=8)
    xcp.wait()

    def edge_group(b, i):
        s = edge_v[b, 0, pl.ds(i * 16, 16)]
        d = edge_v[b, 1, pl.ds(i * 16, 16)]
        xs = plsc.load_gather(x0_v, [lax.shift_right_logical(s, 7),
                                     lax.bitwise_and(s, 127)])
        xd = plsc.load_gather(x0_v, [lax.shift_right_logical(d, 7),
                                     lax.bitwise_and(d, 127)])
        a0 = ax_v[b, i // 8, pl.ds((i % 8) * 16, 16)]
        a1 = ay_v[b, i // 8, pl.ds((i % 8) * 16, 16)]
        diff = xd - xs
        m0 = a0 != 0.0
        m1 = a1 != 0.0
        per0 = jnp.where(m0, diff / jnp.where(m0, a0, onef), zf)
        per1 = jnp.where(m1, diff / jnp.where(m1, a1, onef), zf)
        cnt0 = jnp.where(m0, onef, zf)
        cnt1 = jnp.where(m1, onef, zf)
        plsc.addupdate_scatter(acc_sx, [s], per0)
        plsc.addupdate_scatter(acc_cx, [s], cnt0)
        plsc.addupdate_scatter(acc_sy, [s], per1)
        plsc.addupdate_scatter(acc_cy, [s], cnt1)

    for k in range(SLOTS):
        b = k % 2
        wait(k, b)

        @pl.when(wid + k * NW < N_FULL)
        def _compute():
            def inner(i, c2):
                edge_group(b, i)
                return c2

            lax.fori_loop(0, CHUNK // 16, inner, 0)

        if k + 2 < SLOTS:
            issue(k + 2, b)

    @pl.when(wid == TAIL_WID)
    def _tail():
        gb = N_FULL * CHUNK
        gr = N_FULL * CROWS
        pltpu.sync_copy(edge_hbm.at[:, pl.ds(gb, TAIL)],
                        edge_v.at[0, :, pl.ds(0, TAIL)])
        pltpu.sync_copy(cat_hbm.at[pl.ds(AX_ROW + gr, TAIL // 128), :],
                        ax_v.at[0, pl.ds(0, TAIL // 128), :])
        pltpu.sync_copy(cat_hbm.at[pl.ds(AY_ROW + gr, TAIL // 128), :],
                        ay_v.at[0, pl.ds(0, TAIL // 128), :])

        def inner(i, c2):
            edge_group(0, i)
            return c2

        lax.fori_loop(0, TAIL // 16, inner, 0)

    ob = wid * 4 * NODES_PAD
    outcps = [
        pltpu.make_async_copy(acc, out_hbm.at[pl.ds(ob + j * NODES_PAD,
                                                    NODES_PAD)], semx)
        for j, acc in enumerate((acc_sx, acc_cx, acc_sy, acc_cy))
    ]
    for cp in outcps:
        cp.start()
    for cp in outcps:
        cp.wait()


_sc_partials = functools.partial(
    pl.kernel,
    mesh=plsc.VectorSubcoreMesh(core_axis_name="c", subcore_axis_name="s"),
    compiler_params=pltpu.CompilerParams(needs_layout_passes=False),
    out_type=jax.ShapeDtypeStruct((NW * 4 * NODES_PAD,), jnp.float32),
    scratch_types=[
        pltpu.VMEM((NODES_PAD // 128, 128), jnp.float32),
        pltpu.VMEM((2, 2, CHUNK), jnp.int32),
        pltpu.VMEM((2, CHUNK // 128, 128), jnp.float32),
        pltpu.VMEM((2, CHUNK // 128, 128), jnp.float32),
        pltpu.VMEM((NODES_PAD,), jnp.float32),
        pltpu.VMEM((NODES_PAD,), jnp.float32),
        pltpu.VMEM((NODES_PAD,), jnp.float32),
        pltpu.VMEM((NODES_PAD,), jnp.float32),
        pltpu.SemaphoreType.DMA,
        pltpu.SemaphoreType.DMA,
        pltpu.SemaphoreType.DMA,
    ],
)(_sc_body)


def _tc_reduce(parts_ref, out_ref):
    p = parts_ref[...].reshape(NW, 4, NODES_PAD)
    s = jnp.sum(p, axis=0)
    dx = s[0:1, :] / jnp.maximum(s[1:2, :], 1.0)
    dy = s[2:3, :] / jnp.maximum(s[3:4, :], 1.0)
    out_ref[0:1, :] = dx
    out_ref[1:2, :] = dy


def kernel(x, edge_index, edge_attr):
    cat = jnp.concatenate(
        [x[:, 0], jnp.zeros((NODES_PAD - N_NODES,), jnp.float32),
         edge_attr[:, 0], jnp.zeros((512,), jnp.float32),
         edge_attr[:, 1]]).reshape(-1, 128)
    parts = _sc_partials(cat, edge_index)
    out2 = pl.pallas_call(
        _tc_reduce,
        out_shape=jax.ShapeDtypeStruct((2, NODES_PAD), jnp.float32),
    )(parts)
    return out2[:, :N_NODES].T


# speedup vs baseline: 1.2086x; 1.2086x over previous
import functools

import jax
import jax.numpy as jnp
from jax import lax
from jax.experimental import pallas as pl
from jax.experimental.pallas import tpu as pltpu
from jax.experimental.pallas import tpu_sc as plsc

N_NODES = 10000
N_EDGES = 320000
D_FEAT = 128

NC = 2
NS = 16
NW = NC * NS
CHUNK = 2048
N_CHUNKS = -(-N_EDGES // CHUNK)
N_FULL = N_CHUNKS - 1
TAIL = N_EDGES - N_FULL * CHUNK
TAIL_WID = N_FULL % NW
SLOTS = -(-N_CHUNKS // NW)
NODES_PAD = 10240
X_ROWS = 320


def _sc_body(x0_hbm, edge_hbm, ax_hbm, ay_hbm, out_hbm,
             x0_v, edge_v, ax_v, ay_v,
             acc_sx, acc_cx, acc_sy, acc_cy, sem0, sem1, semx):
    cid = lax.axis_index("c")
    sid = lax.axis_index("s")
    wid = cid * NS + sid
    sems = (sem0, sem1)

    lanes = lax.iota(jnp.int32, 16)
    zf = jnp.zeros((16,), jnp.float32)
    onef = jnp.full((16,), 1.0, jnp.float32)
    col0 = jnp.zeros((16,), jnp.int32)

    def chunk_copies(k, b):
        gb = (wid + k * NW) * CHUNK
        return (
            pltpu.make_async_copy(edge_hbm.at[:, pl.ds(gb, CHUNK)],
                                  edge_v.at[b], sems[b]),
            pltpu.make_async_copy(ax_hbm.at[pl.ds(gb, CHUNK)],
                                  ax_v.at[b], sems[b]),
            pltpu.make_async_copy(ay_hbm.at[pl.ds(gb, CHUNK)],
                                  ay_v.at[b], sems[b]),
        )

    def issue(k, b):
        @pl.when(wid + k * NW < N_FULL)
        def _():
            for cp in chunk_copies(k, b):
                cp.start()

    def wait(k, b):
        @pl.when(wid + k * NW < N_FULL)
        def _():
            for cp in chunk_copies(k, b):
                cp.wait()

    issue(0, 0)
    issue(1, 1)
    xcp = pltpu.make_async_copy(x0_hbm, x0_v.at[pl.ds(0, N_NODES)], semx)
    xcp.start()

    def zero_body(j, carry):
        acc_sx[pl.ds(j * 16, 16)] = zf
        acc_cx[pl.ds(j * 16, 16)] = zf
        acc_sy[pl.ds(j * 16, 16)] = zf
        acc_cy[pl.ds(j * 16, 16)] = zf
        return carry

    lax.fori_loop(0, NODES_PAD // 16, zero_body, 0, unroll=8)
    xcp.wait()

    def edge_group(b, i):
        s = edge_v[b, 0, pl.ds(i * 16, 16)]
        d = edge_v[b, 1, pl.ds(i * 16, 16)]
        xs = plsc.load_gather(x0_v, [s])
        xd = plsc.load_gather(x0_v, [d])
        a0 = ax_v[b, pl.ds(i * 16, 16)]
        a1 = ay_v[b, pl.ds(i * 16, 16)]
        diff = xd - xs
        m0 = a0 != 0.0
        m1 = a1 != 0.0
        per0 = jnp.where(m0, diff / jnp.where(m0, a0, onef), zf)
        per1 = jnp.where(m1, diff / jnp.where(m1, a1, onef), zf)
        cnt0 = jnp.where(m0, onef, zf)
        cnt1 = jnp.where(m1, onef, zf)
        plsc.addupdate_scatter(acc_sx, [s], per0)
        plsc.addupdate_scatter(acc_cx, [s], cnt0)
        plsc.addupdate_scatter(acc_sy, [s], per1)
        plsc.addupdate_scatter(acc_cy, [s], cnt1)

    for k in range(SLOTS):
        b = k % 2
        wait(k, b)

        @pl.when(wid + k * NW < N_FULL)
        def _compute():
            def inner(i, c2):
                edge_group(b, i)
                return c2

            lax.fori_loop(0, CHUNK // 16, inner, 0)

        if k + 2 < SLOTS:
            issue(k + 2, b)

    @pl.when(wid == TAIL_WID)
    def _tail():
        gb = N_FULL * CHUNK
        pltpu.sync_copy(edge_hbm.at[:, pl.ds(gb, TAIL)],
                        edge_v.at[0, :, pl.ds(0, TAIL)])
        pltpu.sync_copy(ax_hbm.at[pl.ds(gb, TAIL)],
                        ax_v.at[0, pl.ds(0, TAIL)])
        pltpu.sync_copy(ay_hbm.at[pl.ds(gb, TAIL)],
                        ay_v.at[0, pl.ds(0, TAIL)])

        def inner(i, c2):
            edge_group(0, i)
            return c2

        lax.fori_loop(0, TAIL // 16, inner, 0)

    ob = wid * 4 * NODES_PAD
    outcps = [
        pltpu.make_async_copy(acc, out_hbm.at[pl.ds(ob + j * NODES_PAD,
                                                    NODES_PAD)], semx)
        for j, acc in enumerate((acc_sx, acc_cx, acc_sy, acc_cy))
    ]
    for cp in outcps:
        cp.start()
    for cp in outcps:
        cp.wait()


_sc_partials = functools.partial(
    pl.kernel,
    mesh=plsc.VectorSubcoreMesh(core_axis_name="c", subcore_axis_name="s"),
    compiler_params=pltpu.CompilerParams(needs_layout_passes=False),
    out_type=jax.ShapeDtypeStruct((NW * 4 * NODES_PAD,), jnp.float32),
    scratch_types=[
        pltpu.VMEM((NODES_PAD,), jnp.float32),
        pltpu.VMEM((2, 2, CHUNK), jnp.int32),
        pltpu.VMEM((2, CHUNK), jnp.float32),
        pltpu.VMEM((2, CHUNK), jnp.float32),
        pltpu.VMEM((NODES_PAD,), jnp.float32),
        pltpu.VMEM((NODES_PAD,), jnp.float32),
        pltpu.VMEM((NODES_PAD,), jnp.float32),
        pltpu.VMEM((NODES_PAD,), jnp.float32),
        pltpu.SemaphoreType.DMA,
        pltpu.SemaphoreType.DMA,
        pltpu.SemaphoreType.DMA,
    ],
)(_sc_body)


def _tc_reduce(parts_ref, out_ref):
    p = parts_ref[...].reshape(NW, 4, NODES_PAD)
    s = jnp.sum(p, axis=0)
    dx = s[0:1, :] / jnp.maximum(s[1:2, :], 1.0)
    dy = s[2:3, :] / jnp.maximum(s[3:4, :], 1.0)
    out_ref[0:1, :] = dx
    out_ref[1:2, :] = dy


def kernel(x, edge_index, edge_attr):
    x0 = x[:, 0]
    ax = edge_attr[:, 0]
    ay = edge_attr[:, 1]
    parts = _sc_partials(x0, edge_index, ax, ay)
    out2 = pl.pallas_call(
        _tc_reduce,
        out_shape=jax.ShapeDtypeStruct((2, NODES_PAD), jnp.float32),
    )(parts)
    return out2[:, :N_NODES].T


# speedup vs baseline: 1.2150x; 1.0053x over previous
import functools

import jax
import jax.numpy as jnp
from jax import lax
from jax.experimental import pallas as pl
from jax.experimental.pallas import tpu as pltpu
from jax.experimental.pallas import tpu_sc as plsc

N_NODES = 10000
N_EDGES = 320000
D_FEAT = 128

NC = 2
NS = 16
NW = NC * NS
CHUNK = 2048
N_CHUNKS = -(-N_EDGES // CHUNK)
N_FULL = N_CHUNKS - 1
TAIL = N_EDGES - N_FULL * CHUNK
TAIL_WID = N_FULL % NW
SLOTS = -(-N_CHUNKS // NW)
NODES_PAD = 10240
X_ROWS = 320


def _sc_body(x0_hbm, edge_hbm, ax_hbm, ay_hbm, out_hbm,
             x0_v, e0, e1, e2, e3, e4, ax0, ax1, ax2, ax3, ax4,
             ay0, ay1, ay2, ay3, ay4,
             acc_sx, acc_cx, acc_sy, acc_cy,
             sem0, sem1, sem2, sem3, sem4, semx):
    cid = lax.axis_index("c")
    sid = lax.axis_index("s")
    wid = cid * NS + sid
    sems = (sem0, sem1, sem2, sem3, sem4)
    edge_v = (e0, e1, e2, e3, e4)
    ax_v = (ax0, ax1, ax2, ax3, ax4)
    ay_v = (ay0, ay1, ay2, ay3, ay4)

    lanes = lax.iota(jnp.int32, 16)
    zf = jnp.zeros((16,), jnp.float32)
    onef = jnp.full((16,), 1.0, jnp.float32)
    col0 = jnp.zeros((16,), jnp.int32)

    def chunk_copies(k, b):
        gb = (wid + k * NW) * CHUNK
        return (
            pltpu.make_async_copy(edge_hbm.at[:, pl.ds(gb, CHUNK)],
                                  edge_v[b], sems[b]),
            pltpu.make_async_copy(ax_hbm.at[pl.ds(gb, CHUNK)],
                                  ax_v[b], sems[b]),
            pltpu.make_async_copy(ay_hbm.at[pl.ds(gb, CHUNK)],
                                  ay_v[b], sems[b]),
        )

    def issue(k, b):
        @pl.when(wid + k * NW < N_FULL)
        def _():
            for cp in chunk_copies(k, b):
                cp.start()

    def wait(k, b):
        @pl.when(wid + k * NW < N_FULL)
        def _():
            for cp in chunk_copies(k, b):
                cp.wait()

    for kk in range(SLOTS):
        issue(kk, kk)
    xcp = pltpu.make_async_copy(x0_hbm, x0_v.at[pl.ds(0, N_NODES)], semx)
    xcp.start()

    def zero_body(j, carry):
        acc_sx[pl.ds(j * 16, 16)] = zf
        acc_cx[pl.ds(j * 16, 16)] = zf
        acc_sy[pl.ds(j * 16, 16)] = zf
        acc_cy[pl.ds(j * 16, 16)] = zf
        return carry

    lax.fori_loop(0, NODES_PAD // 16, zero_body, 0, unroll=8)
    xcp.wait()

    def edge_group(b, i):
        s = edge_v[b][0, pl.ds(i * 16, 16)]
        d = edge_v[b][1, pl.ds(i * 16, 16)]
        xs = plsc.load_gather(x0_v, [s])
        xd = plsc.load_gather(x0_v, [d])
        a0 = ax_v[b][pl.ds(i * 16, 16)]
        a1 = ay_v[b][pl.ds(i * 16, 16)]
        diff = xd - xs
        m0 = a0 != 0.0
        m1 = a1 != 0.0
        per0 = jnp.where(m0, diff / jnp.where(m0, a0, onef), zf)
        per1 = jnp.where(m1, diff / jnp.where(m1, a1, onef), zf)
        cnt0 = jnp.where(m0, onef, zf)
        cnt1 = jnp.where(m1, onef, zf)
        plsc.addupdate_scatter(acc_sx, [s], per0)
        plsc.addupdate_scatter(acc_cx, [s], cnt0)
        plsc.addupdate_scatter(acc_sy, [s], per1)
        plsc.addupdate_scatter(acc_cy, [s], cnt1)

    for k in range(SLOTS):
        wait(k, k)

        @pl.when(wid + k * NW < N_FULL)
        def _compute():
            def inner(i, c2):
                edge_group(k, i)
                return c2

            lax.fori_loop(0, CHUNK // 16, inner, 0)

    @pl.when(wid == TAIL_WID)
    def _tail():
        gb = N_FULL * CHUNK
        pltpu.sync_copy(edge_hbm.at[:, pl.ds(gb, TAIL)],
                        edge_v[0].at[:, pl.ds(0, TAIL)])
        pltpu.sync_copy(ax_hbm.at[pl.ds(gb, TAIL)],
                        ax_v[0].at[pl.ds(0, TAIL)])
        pltpu.sync_copy(ay_hbm.at[pl.ds(gb, TAIL)],
                        ay_v[0].at[pl.ds(0, TAIL)])

        def inner(i, c2):
            edge_group(0, i)
            return c2

        lax.fori_loop(0, TAIL // 16, inner, 0)

    ob = wid * 4 * NODES_PAD
    outcps = [
        pltpu.make_async_copy(acc, out_hbm.at[pl.ds(ob + j * NODES_PAD,
                                                    NODES_PAD)], semx)
        for j, acc in enumerate((acc_sx, acc_cx, acc_sy, acc_cy))
    ]
    for cp in outcps:
        cp.start()
    for cp in outcps:
        cp.wait()


_sc_partials = functools.partial(
    pl.kernel,
    mesh=plsc.VectorSubcoreMesh(core_axis_name="c", subcore_axis_name="s"),
    compiler_params=pltpu.CompilerParams(needs_layout_passes=False),
    out_type=jax.ShapeDtypeStruct((NW * 4 * NODES_PAD,), jnp.float32),
    scratch_types=[
        pltpu.VMEM((NODES_PAD,), jnp.float32),
    ] + [pltpu.VMEM((2, CHUNK), jnp.int32)] * SLOTS
      + [pltpu.VMEM((CHUNK,), jnp.float32)] * SLOTS
      + [pltpu.VMEM((CHUNK,), jnp.float32)] * SLOTS
      + [
        pltpu.VMEM((NODES_PAD,), jnp.float32),
        pltpu.VMEM((NODES_PAD,), jnp.float32),
        pltpu.VMEM((NODES_PAD,), jnp.float32),
        pltpu.VMEM((NODES_PAD,), jnp.float32),
        pltpu.SemaphoreType.DMA,
        pltpu.SemaphoreType.DMA,
        pltpu.SemaphoreType.DMA,
        pltpu.SemaphoreType.DMA,
        pltpu.SemaphoreType.DMA,
        pltpu.SemaphoreType.DMA,
    ],
)(_sc_body)


def _tc_reduce(parts_ref, out_ref):
    p = parts_ref[...].reshape(NW, 4, NODES_PAD)
    s = jnp.sum(p, axis=0)
    dx = s[0:1, :] / jnp.maximum(s[1:2, :], 1.0)
    dy = s[2:3, :] / jnp.maximum(s[3:4, :], 1.0)
    out_ref[0:1, :] = dx
    out_ref[1:2, :] = dy


def kernel(x, edge_index, edge_attr):
    x0 = x[:, 0]
    ax = edge_attr[:, 0]
    ay = edge_attr[:, 1]
    parts = _sc_partials(x0, edge_index, ax, ay)
    out2 = pl.pallas_call(
        _tc_reduce,
        out_shape=jax.ShapeDtypeStruct((2, NODES_PAD), jnp.float32),
    )(parts)
    return out2[:, :N_NODES].T
